# baseline (speedup 1.0000x reference)
import functools

import numpy as np
import jax
import jax.numpy as jnp
from jax import lax
from jax.experimental import pallas as pl
from jax.experimental.pallas import tpu as pltpu
from jax.experimental.pallas import tpu_sc as plsc

PAD = 0
V = 1000000
D = 32
L = 16
IB = 128

_INFO = plsc.get_sparse_core_info()
NC, NS = _INFO.num_cores, _INFO.num_subcores
NW = NC * NS


def _lane_min(v):
    dnums = lax.GatherDimensionNumbers(
        offset_dims=(), collapsed_slice_dims=(0,), start_index_map=(0,)
    )
    for sh in (8, 4, 2, 1):
        perm = (lax.iota(jnp.int32, L) + sh) % L
        rot = lax.gather(
            v, perm[:, None], dnums, (1,),
            mode=lax.GatherScatterMode.PROMISE_IN_BOUNDS,
        )
        v = jnp.minimum(v, rot)
    return v[0]


def _gather_rows(table, idx, B, C):
    b_per_w = B // NW
    n_chunks = b_per_w // C
    gathers_per_chunk = C // IB
    mesh = plsc.VectorSubcoreMesh(core_axis_name="c", subcore_axis_name="s")

    @functools.partial(
        pl.kernel,
        mesh=mesh,
        out_type=jax.ShapeDtypeStruct((B, D), jnp.float32),
        compiler_params=pltpu.CompilerParams(
            needs_layout_passes=False, use_tc_tiling_on_sc=False
        ),
        scratch_types=[
            pltpu.VMEM((C,), jnp.int32),
            pltpu.VMEM((C, D), jnp.float32),
            pltpu.SemaphoreType.DMA,
        ],
    )
    def k(table_hbm, idx_hbm, out_hbm, idx_v, rows_v, sem):
        wid = lax.axis_index("s") * NC + lax.axis_index("c")
        base = wid * b_per_w
        zeros = jnp.zeros((L,), jnp.float32)

        def chunk_body(g, _):
            off = pl.multiple_of(base + g * C, C)
            pltpu.sync_copy(idx_hbm.at[pl.ds(off, C)], idx_v)
            for j in range(gathers_per_chunk):
                pltpu.async_copy(
                    table_hbm.at[idx_v.at[pl.ds(j * IB, IB)]],
                    rows_v.at[pl.ds(j * IB, IB)],
                    sem,
                )

            def min_body(i, acc):
                return jnp.minimum(acc, idx_v[pl.ds(i * L, L)])

            acc = lax.fori_loop(
                0, C // L, min_body, jnp.full((L,), 2**30, jnp.int32),
                unroll=False,
            )
            min_idx = _lane_min(acc)

            for j in range(gathers_per_chunk):
                pltpu.make_async_copy(
                    table_hbm.at[idx_v.at[pl.ds(j * IB, IB)]],
                    rows_v.at[pl.ds(j * IB, IB)],
                    sem,
                ).wait()

            @pl.when(min_idx == PAD)
            def _():
                def fix_body(i, _):
                    iv = idx_v[pl.ds(i * L, L)]
                    m = iv == PAD
                    rowpos = lax.iota(jnp.int32, L) + i * L
                    for j in range(D):
                        plsc.store_scatter(
                            rows_v,
                            [rowpos, jnp.full((L,), j, jnp.int32)],
                            zeros,
                            mask=m,
                        )
                    return 0

                lax.fori_loop(0, C // L, fix_body, 0, unroll=False)

            pltpu.sync_copy(rows_v, out_hbm.at[pl.ds(off, C)])
            return 0

        lax.fori_loop(0, n_chunks, chunk_body, 0, unroll=False)

    return k(table, idx)


def _retile_tc(rows128, NB, T):
    BB = 128

    def body(x_ref, o_ref):
        x = x_ref[...]
        o_ref[0] = x.reshape(BB, D).T

    return pl.pallas_call(
        body,
        grid=(T, NB // BB),
        in_specs=[
            pl.BlockSpec(
                (BB * D // 128, 128), lambda t, i: (i * T + t, 0)
            ),
        ],
        out_specs=pl.BlockSpec((1, D, BB), lambda t, i: (t, 0, i)),
        out_shape=jax.ShapeDtypeStruct((T, D, NB), jnp.float32),
    )(rows128)


@functools.partial(jax.jit, static_argnums=(2, 3))
def _emb_lookup(weight, idxp, NB, T):
    B = NB * T
    rows = _gather_rows(weight, idxp, B, 1024)
    rows128 = rows.reshape(B * D // 128, 128)
    out_t = _retile_tc(rows128, NB, T)
    return jnp.transpose(out_t, (2, 0, 1))


def kernel(input_, weight):
    NB, T = input_.shape
    idxp = input_.reshape(NB // 128, 128, T).transpose(0, 2, 1).reshape(NB * T)
    return _emb_lookup(weight, idxp, NB, T)


# speedup vs baseline: 4.1864x; 4.1864x over previous
import functools

import numpy as np
import jax
import jax.numpy as jnp
from jax import lax
from jax.experimental import pallas as pl
from jax.experimental.pallas import tpu as pltpu
from jax.experimental.pallas import tpu_sc as plsc

PAD = 0
V = 1000000
D = 32
L = 16
IB = 128

_INFO = plsc.get_sparse_core_info()
NC, NS = _INFO.num_cores, _INFO.num_subcores
NW = NC * NS


def _lane_min(v):
    dnums = lax.GatherDimensionNumbers(
        offset_dims=(), collapsed_slice_dims=(0,), start_index_map=(0,)
    )
    for sh in (8, 4, 2, 1):
        perm = (lax.iota(jnp.int32, L) + sh) % L
        rot = lax.gather(
            v, perm[:, None], dnums, (1,),
            mode=lax.GatherScatterMode.PROMISE_IN_BOUNDS,
        )
        v = jnp.minimum(v, rot)
    return v[0]


def _gather_rows(table, idx, B, C):
    b_per_w = B // NW
    n_chunks = b_per_w // C
    gathers_per_chunk = C // IB
    mesh = plsc.VectorSubcoreMesh(core_axis_name="c", subcore_axis_name="s")

    @functools.partial(
        pl.kernel,
        mesh=mesh,
        out_type=jax.ShapeDtypeStruct((B, D), jnp.float32),
        compiler_params=pltpu.CompilerParams(
            needs_layout_passes=False, use_tc_tiling_on_sc=False
        ),
        scratch_types=[
            pltpu.VMEM((C,), jnp.int32),
            pltpu.VMEM((C, D), jnp.float32),
            pltpu.SemaphoreType.DMA,
        ],
    )
    def k(table_hbm, idx_hbm, out_hbm, idx_v, rows_v, sem):
        wid = lax.axis_index("s") * NC + lax.axis_index("c")
        base = wid * b_per_w
        zeros = jnp.zeros((L,), jnp.float32)

        def chunk_body(g, _):
            off = pl.multiple_of(base + g * C, C)
            pltpu.sync_copy(idx_hbm.at[pl.ds(off, C)], idx_v)
            for j in range(gathers_per_chunk):
                pltpu.async_copy(
                    table_hbm.at[idx_v.at[pl.ds(j * IB, IB)]],
                    rows_v.at[pl.ds(j * IB, IB)],
                    sem,
                )

            def min_body(i, acc):
                return jnp.minimum(acc, idx_v[pl.ds(i * L, L)])

            acc = lax.fori_loop(
                0, C // L, min_body, jnp.full((L,), 2**30, jnp.int32),
                unroll=False,
            )
            min_idx = _lane_min(acc)

            for j in range(gathers_per_chunk):
                pltpu.make_async_copy(
                    table_hbm.at[idx_v.at[pl.ds(j * IB, IB)]],
                    rows_v.at[pl.ds(j * IB, IB)],
                    sem,
                ).wait()

            @pl.when(min_idx == PAD)
            def _():
                def fix_body(i, _):
                    iv = idx_v[pl.ds(i * L, L)]
                    m = iv == PAD
                    rowpos = lax.iota(jnp.int32, L) + i * L
                    for j in range(D):
                        plsc.store_scatter(
                            rows_v,
                            [rowpos, jnp.full((L,), j, jnp.int32)],
                            zeros,
                            mask=m,
                        )
                    return 0

                lax.fori_loop(0, C // L, fix_body, 0, unroll=False)

            pltpu.sync_copy(rows_v, out_hbm.at[pl.ds(off, C)])
            return 0

        lax.fori_loop(0, n_chunks, chunk_body, 0, unroll=False)

    return k(table, idx)


def _retile_tc(rows128, NB, T):
    BB = 128
    rows_per_t = BB * D // 128

    def body(x_ref, o_ref):
        for t in range(T):
            x = x_ref[pl.ds(t * rows_per_t, rows_per_t), :]
            o_ref[t] = x.reshape(BB, D).T

    return pl.pallas_call(
        body,
        grid=(NB // BB,),
        in_specs=[
            pl.BlockSpec((T * rows_per_t, 128), lambda i: (i, 0)),
        ],
        out_specs=pl.BlockSpec((T, D, BB), lambda i: (0, 0, i)),
        out_shape=jax.ShapeDtypeStruct((T, D, NB), jnp.float32),
    )(rows128)


@functools.partial(jax.jit, static_argnums=(2, 3))
def _emb_lookup(weight, idxp, NB, T):
    B = NB * T
    rows = _gather_rows(weight, idxp, B, 1024)
    rows128 = rows.reshape(B * D // 128, 128)
    out_t = _retile_tc(rows128, NB, T)
    return jnp.transpose(out_t, (2, 0, 1))


def kernel(input_, weight):
    NB, T = input_.shape
    idxp = input_.reshape(NB // 128, 128, T).transpose(0, 2, 1).reshape(NB * T)
    return _emb_lookup(weight, idxp, NB, T)
